# Initial kernel scaffold; baseline (speedup 1.0000x reference)
import jax
import jax.numpy as jnp
from jax.experimental import pallas as pl


def kernel(data, images):
    raise NotImplementedError("write your pallas kernel here")


# speedup vs baseline: 28.3458x; 28.3458x over previous
import functools

import jax
import jax.numpy as jnp
from jax import lax
from jax.experimental import pallas as pl
from jax.experimental.pallas import tpu as pltpu
from jax.experimental.pallas import tpu_sc as plsc

_H = 300
_W = 300
_N = 256
_NCLS = 64
_NC = 2
_NSUB = 16
_NW = _NC * _NSUB
_RPT = 10
_WPAD = 304
_LANES = 16
_NCHUNK = _WPAD // _LANES


def _round_half_even(x):
    f = jnp.floor(x)
    fi = f.astype(jnp.int32)
    frac = x - f
    up = (frac > 0.5) | ((frac == 0.5) & ((fi & 1) == 1))
    return fi + up.astype(jnp.int32)


def _prep_body(data_ref, img_ref, bnds_ref, coef_ref):
    data = data_ref[...]
    imgs = img_ref[...]

    x = _round_half_even(data[:, 0:1] * _H)
    y = _round_half_even(data[:, 1:2] * _W)
    h = _round_half_even(data[:, 2:3] * _H)
    w = _round_half_even(data[:, 3:4] * _W)
    x1 = x - (h >> 1)
    x2 = x + ((h + 1) >> 1)
    y1 = y - (w >> 1)
    y2 = y + ((w + 1) >> 1)
    xs = jnp.where(x1 < 0, jnp.maximum(x1 + _H, 0), x1)
    xe = jnp.clip(x2, 0, _H)
    ys = jnp.where(y1 < 0, jnp.maximum(y1 + _W, 0), y1)
    ye = jnp.clip(y2, 0, _W)

    logits = data[:, 5:]
    mx = jnp.max(logits, axis=1, keepdims=True)
    col = lax.broadcasted_iota(jnp.int32, logits.shape, 1)
    cls = jnp.min(jnp.where(logits == mx, col, _NCLS), axis=1, keepdims=True)
    onehot = (col == cls).astype(jnp.float32)
    rgba = jnp.dot(onehot, imgs, preferred_element_type=jnp.float32)
    a = rgba[:, 3:4]
    p = rgba[:, 0:3] * a
    q = 1.0 - a

    d = data[:, 4:5]
    dt = jnp.reshape(data[:, 4], (1, _N))
    i_col = lax.broadcasted_iota(jnp.int32, (_N, _N), 1)
    j_row = lax.broadcasted_iota(jnp.int32, (_N, _N), 0)
    before = (dt < d) | ((dt == d) & (i_col < j_row))
    rank = jnp.sum(before.astype(jnp.int32), axis=1, keepdims=True)
    perm = (jnp.reshape(rank, (1, _N)) == j_row).astype(jnp.float32)

    fb = jnp.concatenate(
        [xs.astype(jnp.float32), xe.astype(jnp.float32),
         ys.astype(jnp.float32), ye.astype(jnp.float32)], axis=1)
    fc = jnp.concatenate([p, q], axis=1)
    sb = jnp.dot(perm, fb, preferred_element_type=jnp.float32)
    sc = jnp.dot(perm, fc, preferred_element_type=jnp.float32)
    pad = jnp.zeros((_N, _LANES - 4), jnp.float32)
    bnds_ref[...] = (jnp.concatenate([sb, pad], axis=1) + 0.5).astype(jnp.int32)
    coef_ref[...] = jnp.concatenate([sc, pad], axis=1)


_prep = pl.pallas_call(
    _prep_body,
    out_shape=(
        jax.ShapeDtypeStruct((_N, _LANES), jnp.int32),
        jax.ShapeDtypeStruct((_N, _LANES), jnp.float32),
    ),
)


def _composite_body(bnds_hbm, coef_hbm, out_hbm, bnds_v, coef_v, canvas):
    wid = lax.axis_index("s") * _NC + lax.axis_index("c")
    pltpu.sync_copy(bnds_hbm, bnds_v)
    pltpu.sync_copy(coef_hbm, coef_v)

    ones16 = jnp.full((_LANES,), 1.0, jnp.float32)
    iota16 = lax.broadcasted_iota(jnp.int32, (_LANES,), 0)

    def _init_row(r, _):
        def _init_chunk(t, _):
            for ch in range(4):
                canvas[ch, r, pl.ds(t * _LANES, _LANES)] = ones16
            return 0
        lax.fori_loop(0, _NCHUNK, _init_chunk, 0)
        return 0
    lax.fori_loop(0, _RPT, _init_row, 0)

    def _step(k, _):
        brow = bnds_v[k]
        crow = coef_v[k]
        xs = brow[0]
        xe = brow[1]
        ys = brow[2]
        ye = brow[3]
        p0 = crow[0]
        p1 = crow[1]
        p2 = crow[2]
        q = crow[3]
        i_lo = jnp.maximum((xs - wid + (_NW - 1)) >> 5, 0)
        i_hi = jnp.minimum((xe - wid + (_NW - 1)) >> 5, _RPT)
        t0 = ys >> 4
        t1 = (ye + (_LANES - 1)) >> 4
        qv = jnp.full((_LANES,), q)
        p0v = jnp.full((_LANES,), p0)
        p1v = jnp.full((_LANES,), p1)
        p2v = jnp.full((_LANES,), p2)

        def _chunk(t, _):
            colv = iota16 + t * _LANES
            msk = (colv >= ys) & (colv < ye)

            def _row(i, _):
                sl = pl.ds(t * _LANES, _LANES)
                v0 = canvas[0, i, sl]
                v1 = canvas[1, i, sl]
                v2 = canvas[2, i, sl]
                canvas[0, i, sl] = jnp.where(msk, v0 * qv + p0v, v0)
                canvas[1, i, sl] = jnp.where(msk, v1 * qv + p1v, v1)
                canvas[2, i, sl] = jnp.where(msk, v2 * qv + p2v, v2)
                return 0

            lax.fori_loop(i_lo, i_hi, _row, 0)
            return 0

        lax.fori_loop(t0, t1, _chunk, 0)
        return 0

    lax.fori_loop(0, _N, _step, 0)

    for ch in range(4):
        pltpu.sync_copy(canvas.at[ch], out_hbm.at[ch, wid])


_composite = functools.partial(
    pl.kernel,
    out_type=jax.ShapeDtypeStruct((4, _NW, _RPT, _WPAD), jnp.float32),
    mesh=plsc.VectorSubcoreMesh(
        core_axis_name="c", subcore_axis_name="s",
        num_cores=_NC, num_subcores=_NSUB),
    scratch_types=[
        pltpu.VMEM((_N, _LANES), jnp.int32),
        pltpu.VMEM((_N, _LANES), jnp.float32),
        pltpu.VMEM((4, _RPT, _WPAD), jnp.float32),
    ],
)(_composite_body)


def kernel(data, images):
    bnds, coef = _prep(data, jnp.reshape(images, (_NCLS, 4)))
    tiles = _composite(bnds, coef)
    img = jnp.transpose(tiles, (0, 2, 1, 3)).reshape(4, _NW * _RPT, _WPAD)
    return img[:, :_H, :_W]


# speedup vs baseline: 30.5485x; 1.0777x over previous
import functools

import jax
import jax.numpy as jnp
from jax import lax
from jax.experimental import pallas as pl
from jax.experimental.pallas import tpu as pltpu
from jax.experimental.pallas import tpu_sc as plsc

_H = 300
_W = 300
_N = 256
_NCLS = 64
_NC = 2
_NSUB = 16
_NW = _NC * _NSUB
_RPT = 10
_WPAD = 304
_LANES = 16
_NCHUNK = _WPAD // _LANES


def _round_half_even(x):
    f = jnp.floor(x)
    fi = f.astype(jnp.int32)
    frac = x - f
    up = (frac > 0.5) | ((frac == 0.5) & ((fi & 1) == 1))
    return fi + up.astype(jnp.int32)


def _prep_body(data_ref, datat_ref, img_ref, bnds_ref, coef_ref):
    data = data_ref[...]
    imgs = img_ref[...]

    x = _round_half_even(data[:, 0:1] * _H)
    y = _round_half_even(data[:, 1:2] * _W)
    h = _round_half_even(data[:, 2:3] * _H)
    w = _round_half_even(data[:, 3:4] * _W)
    x1 = x - (h >> 1)
    x2 = x + ((h + 1) >> 1)
    y1 = y - (w >> 1)
    y2 = y + ((w + 1) >> 1)
    xs = jnp.where(x1 < 0, jnp.maximum(x1 + _H, 0), x1)
    xe = jnp.clip(x2, 0, _H)
    ys = jnp.where(y1 < 0, jnp.maximum(y1 + _W, 0), y1)
    ye = jnp.clip(y2, 0, _W)

    logits = data[:, 5:]
    mx = jnp.max(logits, axis=1, keepdims=True)
    col = lax.broadcasted_iota(jnp.int32, logits.shape, 1)
    cls = jnp.min(jnp.where(logits == mx, col, _NCLS), axis=1, keepdims=True)
    onehot = (col == cls).astype(jnp.float32)
    rgba = jnp.dot(onehot, imgs, preferred_element_type=jnp.float32)
    a = rgba[:, 3:4]
    p = rgba[:, 0:3] * a
    q = 1.0 - a

    d = data[:, 4:5]
    dt = datat_ref[4:5, :]
    i_col = lax.broadcasted_iota(jnp.int32, (_N, _N), 1)
    j_row = lax.broadcasted_iota(jnp.int32, (_N, _N), 0)
    before = (dt < d) | ((dt == d) & (i_col < j_row))
    rank = jnp.sum(before.astype(jnp.int32), axis=1, keepdims=True)
    perm = (jnp.reshape(rank, (1, _N)) == j_row).astype(jnp.float32)

    fb = jnp.concatenate(
        [xs.astype(jnp.float32), xe.astype(jnp.float32),
         ys.astype(jnp.float32), ye.astype(jnp.float32)], axis=1)
    fc = jnp.concatenate([p, q], axis=1)
    sb = jnp.dot(perm, fb, preferred_element_type=jnp.float32)
    sc = jnp.dot(perm, fc, preferred_element_type=jnp.float32)
    pad = jnp.zeros((_N, _LANES - 4), jnp.float32)
    bnds_ref[...] = (jnp.concatenate([sb, pad], axis=1) + 0.5).astype(jnp.int32)
    coef_ref[...] = jnp.concatenate([sc, pad], axis=1)


_prep = pl.pallas_call(
    _prep_body,
    out_shape=(
        jax.ShapeDtypeStruct((_N, _LANES), jnp.int32),
        jax.ShapeDtypeStruct((_N, _LANES), jnp.float32),
    ),
)


def _composite_body(bnds_hbm, coef_hbm, out_hbm, bnds_v, coef_v, canvas, sem):
    wid = lax.axis_index("s") * _NC + lax.axis_index("c")
    pltpu.sync_copy(bnds_hbm, bnds_v)
    pltpu.sync_copy(coef_hbm, coef_v)

    ones16 = jnp.full((_LANES,), 1.0, jnp.float32)
    iota16 = lax.broadcasted_iota(jnp.int32, (_LANES,), 0)

    @plsc.parallel_loop(0, _RPT * _NCHUNK, unroll=4)
    def _init(n):
        for ch in range(4):
            canvas[ch, n // _NCHUNK, pl.ds((n % _NCHUNK) * _LANES, _LANES)] = ones16

    def _step(k, _):
        brow = bnds_v[k]
        crow = coef_v[k]
        xs = brow[0]
        xe = brow[1]
        ys = brow[2]
        ye = brow[3]
        qv = jnp.full((_LANES,), crow[3])
        pv = [jnp.full((_LANES,), crow[ch]) for ch in range(3)]
        i_lo = jnp.maximum((xs - wid + (_NW - 1)) >> 5, 0)
        i_hi = jnp.minimum((xe - wid + (_NW - 1)) >> 5, _RPT)
        t0 = ys >> 4
        t_last = (ye - 1) >> 4

        def _masked_rows(t, msk):
            sl = pl.ds(t * _LANES, _LANES)

            @plsc.parallel_loop(i_lo, i_hi, unroll=2)
            def _row(i):
                for ch in range(3):
                    v = canvas[ch, i, sl]
                    canvas[ch, i, sl] = jnp.where(msk, v * qv + pv[ch], v)

        @pl.when((i_lo < i_hi) & (ys < ye))
        def _nonempty():
            colv = iota16 + t0 * _LANES
            _masked_rows(t0, (colv >= ys) & (colv < ye))

            @plsc.parallel_loop(t0 + 1, t_last)
            def _interior(t):
                sl = pl.ds(t * _LANES, _LANES)

                @plsc.parallel_loop(i_lo, i_hi, unroll=2)
                def _row(i):
                    for ch in range(3):
                        canvas[ch, i, sl] = canvas[ch, i, sl] * qv + pv[ch]

            @pl.when(t_last > t0)
            def _last():
                colv = iota16 + t_last * _LANES
                _masked_rows(t_last, colv < ye)

        return 0

    lax.fori_loop(0, _N, _step, 0)

    for i in range(_RPT):
        for ch in range(4):
            pltpu.make_async_copy(
                canvas.at[ch, i], out_hbm.at[ch, wid + _NW * i], sem).start()
    for i in range(_RPT):
        for ch in range(4):
            pltpu.make_async_copy(
                canvas.at[ch, i], out_hbm.at[ch, wid + _NW * i], sem).wait()


_composite = functools.partial(
    pl.kernel,
    out_type=jax.ShapeDtypeStruct((4, _NW * _RPT, _WPAD), jnp.float32),
    mesh=plsc.VectorSubcoreMesh(
        core_axis_name="c", subcore_axis_name="s",
        num_cores=_NC, num_subcores=_NSUB),
    scratch_types=[
        pltpu.VMEM((_N, _LANES), jnp.int32),
        pltpu.VMEM((_N, _LANES), jnp.float32),
        pltpu.VMEM((4, _RPT, _WPAD), jnp.float32),
        pltpu.SemaphoreType.DMA,
    ],
)(_composite_body)


def kernel(data, images):
    bnds, coef = _prep(data, data.T, jnp.reshape(images, (_NCLS, 4)))
    tiles = _composite(bnds, coef)
    return tiles[:, :_H, :_W]


# speedup vs baseline: 36.6815x; 1.2008x over previous
import functools

import jax
import jax.numpy as jnp
from jax import lax
from jax.experimental import pallas as pl
from jax.experimental.pallas import tpu as pltpu
from jax.experimental.pallas import tpu_sc as plsc

_H = 300
_W = 300
_N = 256
_NCLS = 64
_NC = 2
_NSUB = 16
_NW = _NC * _NSUB
_RPT = 10
_WPAD = 304
_LANES = 16
_NCHUNK = _WPAD // _LANES


def _round_half_even(x):
    f = jnp.floor(x)
    fi = f.astype(jnp.int32)
    frac = x - f
    up = (frac > 0.5) | ((frac == 0.5) & ((fi & 1) == 1))
    return fi + up.astype(jnp.int32)


def _prep_body(data_ref, datat_ref, img_ref, bnds_ref, coef_ref):
    data = data_ref[...]
    imgs = img_ref[...]

    x = _round_half_even(data[:, 0:1] * _H)
    y = _round_half_even(data[:, 1:2] * _W)
    h = _round_half_even(data[:, 2:3] * _H)
    w = _round_half_even(data[:, 3:4] * _W)
    x1 = x - (h >> 1)
    x2 = x + ((h + 1) >> 1)
    y1 = y - (w >> 1)
    y2 = y + ((w + 1) >> 1)
    xs = jnp.where(x1 < 0, jnp.maximum(x1 + _H, 0), x1)
    xe = jnp.clip(x2, 0, _H)
    ys = jnp.where(y1 < 0, jnp.maximum(y1 + _W, 0), y1)
    ye = jnp.clip(y2, 0, _W)

    logits = data[:, 5:]
    mx = jnp.max(logits, axis=1, keepdims=True)
    col = lax.broadcasted_iota(jnp.int32, logits.shape, 1)
    cls = jnp.min(jnp.where(logits == mx, col, _NCLS), axis=1, keepdims=True)
    onehot = (col == cls).astype(jnp.float32)
    rgba = jnp.dot(onehot, imgs, preferred_element_type=jnp.float32)
    a = rgba[:, 3:4]
    p = rgba[:, 0:3] * a
    q = 1.0 - a

    d = data[:, 4:5]
    dt = datat_ref[4:5, :]
    i_col = lax.broadcasted_iota(jnp.int32, (_N, _N), 1)
    j_row = lax.broadcasted_iota(jnp.int32, (_N, _N), 0)
    before = (dt < d) | ((dt == d) & (i_col < j_row))
    rank = jnp.sum(before.astype(jnp.int32), axis=1, keepdims=True)
    perm = (jnp.reshape(rank, (1, _N)) == j_row).astype(jnp.float32)

    fb = jnp.concatenate(
        [xs.astype(jnp.float32), xe.astype(jnp.float32),
         ys.astype(jnp.float32), ye.astype(jnp.float32)], axis=1)
    fc = jnp.concatenate([p, q], axis=1)
    sb = jnp.dot(perm, fb, preferred_element_type=jnp.float32)
    sc = jnp.dot(perm, fc, preferred_element_type=jnp.float32)
    pad = jnp.zeros((_N, _LANES - 4), jnp.float32)
    bnds_ref[...] = (jnp.concatenate([sb, pad], axis=1) + 0.5).astype(jnp.int32)
    coef_ref[...] = jnp.concatenate([sc, pad], axis=1)


_prep = pl.pallas_call(
    _prep_body,
    out_shape=(
        jax.ShapeDtypeStruct((_N, _LANES), jnp.int32),
        jax.ShapeDtypeStruct((_N, _LANES), jnp.float32),
    ),
)


def _composite_body(bnds_hbm, coef_hbm, out_hbm, bnds_v, coef_v, canvas, sem):
    wid = lax.axis_index("s") * _NC + lax.axis_index("c")
    pltpu.sync_copy(bnds_hbm, bnds_v)
    pltpu.sync_copy(coef_hbm, coef_v)

    ones16 = jnp.full((_LANES,), 1.0, jnp.float32)
    iota16 = lax.broadcasted_iota(jnp.int32, (_LANES,), 0)

    @plsc.parallel_loop(0, _RPT * _NCHUNK, unroll=4)
    def _init(n):
        for ch in range(4):
            canvas[ch, n // _NCHUNK, pl.ds((n % _NCHUNK) * _LANES, _LANES)] = ones16

    def _step(k, _):
        brow = bnds_v[k]
        crow = coef_v[k]
        xs = brow[0]
        xe = brow[1]
        ys = brow[2]
        ye = brow[3]
        qv = jnp.full((_LANES,), crow[3])
        pv = [jnp.full((_LANES,), crow[ch]) for ch in range(3)]
        i_lo = jnp.maximum((xs - wid + (_NW - 1)) >> 5, 0)
        i_hi = jnp.minimum((xe - wid + (_NW - 1)) >> 5, _RPT)
        t0 = ys >> 4
        t_last = (ye - 1) >> 4

        def _masked_rows(t, msk, blend):
            sl = pl.ds(t * _LANES, _LANES)

            @plsc.parallel_loop(i_lo, i_hi, unroll=2)
            def _row(i):
                for ch in range(3):
                    v = canvas[ch, i, sl]
                    nv = v * qv + pv[ch] if blend else pv[ch]
                    canvas[ch, i, sl] = jnp.where(msk, nv, v)

        def _edges_and_interior(blend):
            colv = iota16 + t0 * _LANES
            _masked_rows(t0, (colv >= ys) & (colv < ye), blend)

            @plsc.parallel_loop(t0 + 1, t_last)
            def _interior(t):
                sl = pl.ds(t * _LANES, _LANES)

                @plsc.parallel_loop(i_lo, i_hi, unroll=2)
                def _row(i):
                    for ch in range(3):
                        if blend:
                            canvas[ch, i, sl] = canvas[ch, i, sl] * qv + pv[ch]
                        else:
                            canvas[ch, i, sl] = pv[ch]

            @pl.when(t_last > t0)
            def _last():
                colv2 = iota16 + t_last * _LANES
                _masked_rows(t_last, colv2 < ye, blend)

        nonempty = (i_lo < i_hi) & (ys < ye)
        opaque = crow[3] == 0.0

        @pl.when(nonempty & opaque)
        def _paint():
            _edges_and_interior(blend=False)

        @pl.when(nonempty & jnp.logical_not(opaque))
        def _blend():
            _edges_and_interior(blend=True)

        return 0

    lax.fori_loop(0, _N, _step, 0)

    for i in range(_RPT):
        for ch in range(4):
            pltpu.make_async_copy(
                canvas.at[ch, i], out_hbm.at[ch, wid + _NW * i], sem).start()
    for i in range(_RPT):
        for ch in range(4):
            pltpu.make_async_copy(
                canvas.at[ch, i], out_hbm.at[ch, wid + _NW * i], sem).wait()


_composite = functools.partial(
    pl.kernel,
    out_type=jax.ShapeDtypeStruct((4, _NW * _RPT, _WPAD), jnp.float32),
    mesh=plsc.VectorSubcoreMesh(
        core_axis_name="c", subcore_axis_name="s",
        num_cores=_NC, num_subcores=_NSUB),
    scratch_types=[
        pltpu.VMEM((_N, _LANES), jnp.int32),
        pltpu.VMEM((_N, _LANES), jnp.float32),
        pltpu.VMEM((4, _RPT, _WPAD), jnp.float32),
        pltpu.SemaphoreType.DMA,
    ],
)(_composite_body)


def kernel(data, images):
    bnds, coef = _prep(data, data.T, jnp.reshape(images, (_NCLS, 4)))
    tiles = _composite(bnds, coef)
    return tiles[:, :_H, :_W]


# speedup vs baseline: 44.6983x; 1.2185x over previous
import functools

import jax
import jax.numpy as jnp
from jax import lax
from jax.experimental import pallas as pl
from jax.experimental.pallas import tpu as pltpu
from jax.experimental.pallas import tpu_sc as plsc

_H = 300
_W = 300
_N = 256
_NCLS = 64
_NC = 2
_NSUB = 16
_NW = _NC * _NSUB
_RPT = 10
_WPAD = 304
_LANES = 16
_NCHUNK = _WPAD // _LANES


def _round_half_even(x):
    f = jnp.floor(x)
    fi = f.astype(jnp.int32)
    frac = x - f
    up = (frac > 0.5) | ((frac == 0.5) & ((fi & 1) == 1))
    return fi + up.astype(jnp.int32)


def _prep_body(data_ref, datat_ref, img_ref, bnds_ref, coef_ref):
    data = data_ref[...]
    imgs = img_ref[...]

    x = _round_half_even(data[:, 0:1] * _H)
    y = _round_half_even(data[:, 1:2] * _W)
    h = _round_half_even(data[:, 2:3] * _H)
    w = _round_half_even(data[:, 3:4] * _W)
    x1 = x - (h >> 1)
    x2 = x + ((h + 1) >> 1)
    y1 = y - (w >> 1)
    y2 = y + ((w + 1) >> 1)
    xs = jnp.where(x1 < 0, jnp.maximum(x1 + _H, 0), x1)
    xe = jnp.clip(x2, 0, _H)
    ys = jnp.where(y1 < 0, jnp.maximum(y1 + _W, 0), y1)
    ye = jnp.clip(y2, 0, _W)

    logits = data[:, 5:]
    mx = jnp.max(logits, axis=1, keepdims=True)
    col = lax.broadcasted_iota(jnp.int32, logits.shape, 1)
    cls = jnp.min(jnp.where(logits == mx, col, _NCLS), axis=1, keepdims=True)
    onehot = (col == cls).astype(jnp.float32)
    rgba = jnp.dot(onehot, imgs, preferred_element_type=jnp.float32)
    a = rgba[:, 3:4]
    p = rgba[:, 0:3] * a
    q = 1.0 - a

    d = data[:, 4:5]
    dt = datat_ref[4:5, :]
    i_col = lax.broadcasted_iota(jnp.int32, (_N, _N), 1)
    j_row = lax.broadcasted_iota(jnp.int32, (_N, _N), 0)
    before = (dt < d) | ((dt == d) & (i_col < j_row))
    rank = jnp.sum(before.astype(jnp.int32), axis=1, keepdims=True)
    perm = (jnp.reshape(rank, (1, _N)) == j_row).astype(jnp.float32)

    fb = jnp.concatenate(
        [xs.astype(jnp.float32), xe.astype(jnp.float32),
         ys.astype(jnp.float32), ye.astype(jnp.float32)], axis=1)
    fc = jnp.concatenate([p, q], axis=1)
    sb = jnp.dot(perm, fb, preferred_element_type=jnp.float32)
    sc = jnp.dot(perm, fc, preferred_element_type=jnp.float32)
    pad = jnp.zeros((_N, _LANES - 4), jnp.float32)
    bnds_ref[...] = (jnp.concatenate([sb, pad], axis=1) + 0.5).astype(jnp.int32)
    coef_ref[...] = jnp.concatenate([sc, pad], axis=1)


_prep = pl.pallas_call(
    _prep_body,
    out_shape=(
        jax.ShapeDtypeStruct((_N, _LANES), jnp.int32),
        jax.ShapeDtypeStruct((_N, _LANES), jnp.float32),
    ),
)


_CH = _RPT * _WPAD


def _composite_body(bnds_hbm, coef_hbm, out_hbm, bnds_v, coef_v, canvas, sem):
    wid = lax.axis_index("s") * _NC + lax.axis_index("c")
    pltpu.sync_copy(bnds_hbm, bnds_v)
    pltpu.sync_copy(coef_hbm, coef_v)

    ones16 = jnp.full((_LANES,), 1.0, jnp.float32)
    iota16 = lax.broadcasted_iota(jnp.int32, (_LANES,), 0)

    @plsc.parallel_loop(0, 4 * _CH // _LANES, unroll=8)
    def _init(n):
        canvas[pl.ds(n * _LANES, _LANES)] = ones16

    def _step(k, _):
        brow = bnds_v[k]
        xs = brow[0]
        xe = brow[1]
        ys = brow[2]
        ye = brow[3]
        i_lo = jnp.maximum((xs - wid + (_NW - 1)) >> 5, 0)
        i_hi = jnp.minimum((xe - wid + (_NW - 1)) >> 5, _RPT)
        t0 = ys >> 4
        t_last = (ye - 1) >> 4

        @pl.when((i_lo < i_hi) & (ys < ye))
        def _nonempty():
            crow = coef_v[k]
            qv = jnp.full((_LANES,), crow[3])
            pv = [jnp.full((_LANES,), crow[ch]) for ch in range(3)]

            def _edge(t, msk, blend):
                off0 = t * _LANES

                @plsc.parallel_loop(i_lo, i_hi, unroll=2)
                def _row(i):
                    base = off0 + i * _WPAD
                    for ch in range(3):
                        sl = pl.ds(base + ch * _CH, _LANES)
                        v = canvas[sl]
                        nv = v * qv + pv[ch] if blend else pv[ch]
                        canvas[sl] = jnp.where(msk, nv, v)

            def _do(blend):
                colv = iota16 + t0 * _LANES
                _edge(t0, (colv >= ys) & (colv < ye), blend)

                @plsc.parallel_loop(i_lo, i_hi)
                def _rows(i):
                    rb = i * _WPAD

                    @plsc.parallel_loop(t0 + 1, t_last, unroll=2)
                    def _t(t):
                        base = rb + t * _LANES
                        for ch in range(3):
                            sl = pl.ds(base + ch * _CH, _LANES)
                            if blend:
                                canvas[sl] = canvas[sl] * qv + pv[ch]
                            else:
                                canvas[sl] = pv[ch]

                @pl.when(t_last > t0)
                def _last():
                    colv2 = iota16 + t_last * _LANES
                    _edge(t_last, colv2 < ye, blend)

            opaque = crow[3] == 0.0

            @pl.when(opaque)
            def _paint():
                _do(False)

            @pl.when(jnp.logical_not(opaque))
            def _blendp():
                _do(True)

        return 0

    lax.fori_loop(0, _N, _step, 0)

    for i in range(_RPT):
        for ch in range(4):
            pltpu.make_async_copy(
                canvas.at[pl.ds(ch * _CH + i * _WPAD, _WPAD)],
                out_hbm.at[pl.ds((ch * _NW * _RPT + wid + _NW * i) * _WPAD,
                                 _WPAD)], sem).start()
    for i in range(_RPT):
        for ch in range(4):
            pltpu.make_async_copy(
                canvas.at[pl.ds(ch * _CH + i * _WPAD, _WPAD)],
                out_hbm.at[pl.ds((ch * _NW * _RPT + wid + _NW * i) * _WPAD,
                                 _WPAD)], sem).wait()


_composite = functools.partial(
    pl.kernel,
    out_type=jax.ShapeDtypeStruct((4 * _NW * _RPT * _WPAD,), jnp.float32),
    mesh=plsc.VectorSubcoreMesh(
        core_axis_name="c", subcore_axis_name="s",
        num_cores=_NC, num_subcores=_NSUB),
    scratch_types=[
        pltpu.VMEM((_N, _LANES), jnp.int32),
        pltpu.VMEM((_N, _LANES), jnp.float32),
        pltpu.VMEM((4 * _CH,), jnp.float32),
        pltpu.SemaphoreType.DMA,
    ],
)(_composite_body)


def kernel(data, images):
    bnds, coef = _prep(data, data.T, jnp.reshape(images, (_NCLS, 4)))
    tiles = _composite(bnds, coef).reshape(4, _NW * _RPT, _WPAD)
    return tiles[:, :_H, :_W]


# speedup vs baseline: 44.7977x; 1.0022x over previous
import functools

import jax
import jax.numpy as jnp
from jax import lax
from jax.experimental import pallas as pl
from jax.experimental.pallas import tpu as pltpu
from jax.experimental.pallas import tpu_sc as plsc

_H = 300
_W = 300
_N = 256
_NCLS = 64
_NC = 2
_NSUB = 16
_NW = _NC * _NSUB
_RPT = 10
_WPAD = 304
_LANES = 16
_NCHUNK = _WPAD // _LANES


def _round_half_even(x):
    f = jnp.floor(x)
    fi = f.astype(jnp.int32)
    frac = x - f
    up = (frac > 0.5) | ((frac == 0.5) & ((fi & 1) == 1))
    return fi + up.astype(jnp.int32)


def _prep_body(data_ref, img_ref, bnds_ref, coef_ref):
    data = data_ref[...]
    imgs = img_ref[...]

    x = _round_half_even(data[:, 0:1] * _H)
    y = _round_half_even(data[:, 1:2] * _W)
    h = _round_half_even(data[:, 2:3] * _H)
    w = _round_half_even(data[:, 3:4] * _W)
    x1 = x - (h >> 1)
    x2 = x + ((h + 1) >> 1)
    y1 = y - (w >> 1)
    y2 = y + ((w + 1) >> 1)
    xs = jnp.where(x1 < 0, jnp.maximum(x1 + _H, 0), x1)
    xe = jnp.clip(x2, 0, _H)
    ys = jnp.where(y1 < 0, jnp.maximum(y1 + _W, 0), y1)
    ye = jnp.clip(y2, 0, _W)

    logits = data[:, 5:]
    mx = jnp.max(logits, axis=1, keepdims=True)
    col = lax.broadcasted_iota(jnp.int32, logits.shape, 1)
    cls = jnp.min(jnp.where(logits == mx, col, _NCLS), axis=1, keepdims=True)
    onehot = (col == cls).astype(jnp.float32)
    rgba = jnp.dot(onehot, imgs, preferred_element_type=jnp.float32)
    a = rgba[:, 3:4]
    p = rgba[:, 0:3] * a
    q = 1.0 - a

    d = data[:, 4:5]
    dt = lax.transpose(d, (1, 0))
    i_col = lax.broadcasted_iota(jnp.int32, (_N, _N), 1)
    j_row = lax.broadcasted_iota(jnp.int32, (_N, _N), 0)
    before = (dt < d) | ((dt == d) & (i_col < j_row))
    rank = jnp.sum(before.astype(jnp.int32), axis=1, keepdims=True)
    perm = (jnp.reshape(rank, (1, _N)) == j_row).astype(jnp.float32)

    fb = jnp.concatenate(
        [xs.astype(jnp.float32), xe.astype(jnp.float32),
         ys.astype(jnp.float32), ye.astype(jnp.float32)], axis=1)
    fc = jnp.concatenate([p, q], axis=1)
    sb = jnp.dot(perm, fb, preferred_element_type=jnp.float32)
    sc = jnp.dot(perm, fc, preferred_element_type=jnp.float32)
    pad = jnp.zeros((_N, _LANES - 4), jnp.float32)
    bnds_ref[...] = (jnp.concatenate([sb, pad], axis=1) + 0.5).astype(jnp.int32)
    coef_ref[...] = jnp.concatenate([sc, pad], axis=1)


_prep = pl.pallas_call(
    _prep_body,
    out_shape=(
        jax.ShapeDtypeStruct((_N, _LANES), jnp.int32),
        jax.ShapeDtypeStruct((_N, _LANES), jnp.float32),
    ),
)


_CH = _RPT * _WPAD


def _composite_body(bnds_hbm, coef_hbm, out_hbm, bnds_v, coef_v, canvas,
                    sem, insem):
    wid = lax.axis_index("s") * _NC + lax.axis_index("c")
    pltpu.make_async_copy(bnds_hbm, bnds_v, insem).start()
    pltpu.make_async_copy(coef_hbm, coef_v, insem).start()

    ones16 = jnp.full((_LANES,), 1.0, jnp.float32)
    iota16 = lax.broadcasted_iota(jnp.int32, (_LANES,), 0)

    @plsc.parallel_loop(0, 4 * _CH // _LANES, unroll=8)
    def _init(n):
        canvas[pl.ds(n * _LANES, _LANES)] = ones16

    for i in range(_RPT):
        pltpu.make_async_copy(
            canvas.at[pl.ds(3 * _CH + i * _WPAD, _WPAD)],
            out_hbm.at[pl.ds((3 * _NW * _RPT + wid + _NW * i) * _WPAD,
                             _WPAD)], sem).start()

    pltpu.make_async_copy(bnds_hbm, bnds_v, insem).wait()
    pltpu.make_async_copy(coef_hbm, coef_v, insem).wait()

    def _step(k, _):
        brow = bnds_v[k]
        xs = brow[0]
        xe = brow[1]
        ys = brow[2]
        ye = brow[3]
        i_lo = jnp.maximum((xs - wid + (_NW - 1)) >> 5, 0)
        i_hi = jnp.minimum((xe - wid + (_NW - 1)) >> 5, _RPT)
        t0 = ys >> 4
        t_last = (ye - 1) >> 4

        @pl.when((i_lo < i_hi) & (ys < ye))
        def _nonempty():
            crow = coef_v[k]
            qv = jnp.full((_LANES,), crow[3])
            pv = [jnp.full((_LANES,), crow[ch]) for ch in range(3)]

            def _edge(t, msk, blend):
                off0 = t * _LANES

                @plsc.parallel_loop(i_lo, i_hi, unroll=2)
                def _row(i):
                    base = off0 + i * _WPAD
                    for ch in range(3):
                        sl = pl.ds(base + ch * _CH, _LANES)
                        v = canvas[sl]
                        nv = v * qv + pv[ch] if blend else pv[ch]
                        canvas[sl] = jnp.where(msk, nv, v)

            def _do(blend):
                colv = iota16 + t0 * _LANES
                _edge(t0, (colv >= ys) & (colv < ye), blend)

                @plsc.parallel_loop(i_lo, i_hi)
                def _rows(i):
                    rb = i * _WPAD

                    @plsc.parallel_loop(t0 + 1, t_last, unroll=4)
                    def _t(t):
                        base = rb + t * _LANES
                        for ch in range(3):
                            sl = pl.ds(base + ch * _CH, _LANES)
                            if blend:
                                canvas[sl] = canvas[sl] * qv + pv[ch]
                            else:
                                canvas[sl] = pv[ch]

                @pl.when(t_last > t0)
                def _last():
                    colv2 = iota16 + t_last * _LANES
                    _edge(t_last, colv2 < ye, blend)

            opaque = crow[3] == 0.0

            @pl.when(opaque)
            def _paint():
                _do(False)

            @pl.when(jnp.logical_not(opaque))
            def _blendp():
                _do(True)

        return 0

    lax.fori_loop(0, _N, _step, 0)

    for i in range(_RPT):
        for ch in range(3):
            pltpu.make_async_copy(
                canvas.at[pl.ds(ch * _CH + i * _WPAD, _WPAD)],
                out_hbm.at[pl.ds((ch * _NW * _RPT + wid + _NW * i) * _WPAD,
                                 _WPAD)], sem).start()
    for i in range(_RPT):
        for ch in range(4):
            pltpu.make_async_copy(
                canvas.at[pl.ds(ch * _CH + i * _WPAD, _WPAD)],
                out_hbm.at[pl.ds((ch * _NW * _RPT + wid + _NW * i) * _WPAD,
                                 _WPAD)], sem).wait()


_composite = functools.partial(
    pl.kernel,
    out_type=jax.ShapeDtypeStruct((4 * _NW * _RPT * _WPAD,), jnp.float32),
    mesh=plsc.VectorSubcoreMesh(
        core_axis_name="c", subcore_axis_name="s",
        num_cores=_NC, num_subcores=_NSUB),
    scratch_types=[
        pltpu.VMEM((_N, _LANES), jnp.int32),
        pltpu.VMEM((_N, _LANES), jnp.float32),
        pltpu.VMEM((4 * _CH,), jnp.float32),
        pltpu.SemaphoreType.DMA,
        pltpu.SemaphoreType.DMA,
    ],
)(_composite_body)


def kernel(data, images):
    bnds, coef = _prep(data, jnp.reshape(images, (_NCLS, 4)))
    tiles = _composite(bnds, coef).reshape(4, _NW * _RPT, _WPAD)
    return tiles[:, :_H, :_W]


# speedup vs baseline: 49.4213x; 1.1032x over previous
import functools

import jax
import jax.numpy as jnp
from jax import lax
from jax.experimental import pallas as pl
from jax.experimental.pallas import tpu as pltpu
from jax.experimental.pallas import tpu_sc as plsc

_H = 300
_W = 300
_N = 256
_NCLS = 64
_NC = 2
_NSUB = 16
_NW = _NC * _NSUB
_RPT = 10
_WPAD = 304
_LANES = 16
_NCHUNK = _WPAD // _LANES


def _round_half_even(x):
    f = jnp.floor(x)
    fi = f.astype(jnp.int32)
    frac = x - f
    up = (frac > 0.5) | ((frac == 0.5) & ((fi & 1) == 1))
    return fi + up.astype(jnp.int32)


def _prep_body(data_ref, img_ref, bnds_ref, coef_ref):
    data = data_ref[...]
    imgs = img_ref[...]

    x = _round_half_even(data[:, 0:1] * _H)
    y = _round_half_even(data[:, 1:2] * _W)
    h = _round_half_even(data[:, 2:3] * _H)
    w = _round_half_even(data[:, 3:4] * _W)
    x1 = x - (h >> 1)
    x2 = x + ((h + 1) >> 1)
    y1 = y - (w >> 1)
    y2 = y + ((w + 1) >> 1)
    xs = jnp.where(x1 < 0, jnp.maximum(x1 + _H, 0), x1)
    xe = jnp.clip(x2, 0, _H)
    ys = jnp.where(y1 < 0, jnp.maximum(y1 + _W, 0), y1)
    ye = jnp.clip(y2, 0, _W)

    logits = data[:, 5:]
    mx = jnp.max(logits, axis=1, keepdims=True)
    col = lax.broadcasted_iota(jnp.int32, logits.shape, 1)
    cls = jnp.min(jnp.where(logits == mx, col, _NCLS), axis=1, keepdims=True)
    onehot = (col == cls).astype(jnp.float32)
    rgba = jnp.dot(onehot, imgs, preferred_element_type=jnp.float32)
    a = rgba[:, 3:4]
    p = rgba[:, 0:3] * a
    q = 1.0 - a

    d = data[:, 4:5]
    dt = lax.transpose(d, (1, 0))
    i_col = lax.broadcasted_iota(jnp.int32, (_N, _N), 1)
    j_row = lax.broadcasted_iota(jnp.int32, (_N, _N), 0)
    before = (dt < d) | ((dt == d) & (i_col < j_row))
    rank = jnp.sum(before.astype(jnp.int32), axis=1, keepdims=True)
    perm = (lax.transpose(rank, (1, 0)) == j_row).astype(jnp.float32)

    fb = jnp.concatenate(
        [xs.astype(jnp.float32), xe.astype(jnp.float32),
         ys.astype(jnp.float32), ye.astype(jnp.float32)], axis=1)
    fc = jnp.concatenate([p, q], axis=1)
    sb = jnp.dot(perm, fb, preferred_element_type=jnp.float32)
    sc = jnp.dot(perm, fc, preferred_element_type=jnp.float32)
    pad = jnp.zeros((_N, _LANES - 4), jnp.float32)
    bnds_ref[...] = (jnp.concatenate([sb, pad], axis=1) + 0.5).astype(jnp.int32)
    coef_ref[...] = jnp.concatenate([sc, pad], axis=1)


_prep = pl.pallas_call(
    _prep_body,
    out_shape=(
        jax.ShapeDtypeStruct((_N, _LANES), jnp.int32),
        jax.ShapeDtypeStruct((_N, _LANES), jnp.float32),
    ),
)


_CH = _RPT * _WPAD


def _composite_body(bnds_hbm, coef_hbm, out_hbm, bnds_v, coef_v, canvas,
                    sem, insem):
    wid = lax.axis_index("s") * _NC + lax.axis_index("c")
    pltpu.make_async_copy(bnds_hbm, bnds_v, insem).start()
    pltpu.make_async_copy(coef_hbm, coef_v, insem).start()

    ones16 = jnp.full((_LANES,), 1.0, jnp.float32)
    iota16 = lax.broadcasted_iota(jnp.int32, (_LANES,), 0)

    @plsc.parallel_loop(0, 4 * _CH // _LANES, unroll=8)
    def _init(n):
        canvas[pl.ds(n * _LANES, _LANES)] = ones16

    for i in range(_RPT):
        pltpu.make_async_copy(
            canvas.at[pl.ds(3 * _CH + i * _WPAD, _WPAD)],
            out_hbm.at[pl.ds((3 * _NW * _RPT + wid + _NW * i) * _WPAD,
                             _WPAD)], sem).start()

    pltpu.make_async_copy(bnds_hbm, bnds_v, insem).wait()
    pltpu.make_async_copy(coef_hbm, coef_v, insem).wait()

    def _step(k, _):
        brow = bnds_v[k]
        xs = brow[0]
        xe = brow[1]
        ys = brow[2]
        ye = brow[3]
        i_lo = jnp.maximum((xs - wid + (_NW - 1)) >> 5, 0)
        i_hi = jnp.minimum((xe - wid + (_NW - 1)) >> 5, _RPT)
        t0 = ys >> 4
        t_last = (ye - 1) >> 4

        @pl.when((i_lo < i_hi) & (ys < ye))
        def _nonempty():
            crow = coef_v[k]
            qv = jnp.full((_LANES,), crow[3])
            pv = [jnp.full((_LANES,), crow[ch]) for ch in range(3)]

            def _edge(t, msk, blend):
                off0 = t * _LANES

                @plsc.parallel_loop(i_lo, i_hi, unroll=2)
                def _row(i):
                    base = off0 + i * _WPAD
                    for ch in range(3):
                        sl = pl.ds(base + ch * _CH, _LANES)
                        v = canvas[sl]
                        nv = v * qv + pv[ch] if blend else pv[ch]
                        canvas[sl] = jnp.where(msk, nv, v)

            def _do(blend):
                colv = iota16 + t0 * _LANES
                _edge(t0, (colv >= ys) & (colv < ye), blend)

                @plsc.parallel_loop(i_lo, i_hi)
                def _rows(i):
                    rb = i * _WPAD

                    @plsc.parallel_loop(t0 + 1, t_last, unroll=4)
                    def _t(t):
                        base = rb + t * _LANES
                        for ch in range(3):
                            sl = pl.ds(base + ch * _CH, _LANES)
                            if blend:
                                canvas[sl] = canvas[sl] * qv + pv[ch]
                            else:
                                canvas[sl] = pv[ch]

                @pl.when(t_last > t0)
                def _last():
                    colv2 = iota16 + t_last * _LANES
                    _edge(t_last, colv2 < ye, blend)

            opaque = crow[3] == 0.0

            @pl.when(opaque)
            def _paint():
                _do(False)

            @pl.when(jnp.logical_not(opaque))
            def _blendp():
                _do(True)

        return 0

    lax.fori_loop(0, _N, _step, 0)

    for i in range(_RPT):
        for ch in range(3):
            pltpu.make_async_copy(
                canvas.at[pl.ds(ch * _CH + i * _WPAD, _WPAD)],
                out_hbm.at[pl.ds((ch * _NW * _RPT + wid + _NW * i) * _WPAD,
                                 _WPAD)], sem).start()
    for i in range(_RPT):
        for ch in range(4):
            pltpu.make_async_copy(
                canvas.at[pl.ds(ch * _CH + i * _WPAD, _WPAD)],
                out_hbm.at[pl.ds((ch * _NW * _RPT + wid + _NW * i) * _WPAD,
                                 _WPAD)], sem).wait()


_composite = functools.partial(
    pl.kernel,
    out_type=jax.ShapeDtypeStruct((4 * _NW * _RPT * _WPAD,), jnp.float32),
    mesh=plsc.VectorSubcoreMesh(
        core_axis_name="c", subcore_axis_name="s",
        num_cores=_NC, num_subcores=_NSUB),
    scratch_types=[
        pltpu.VMEM((_N, _LANES), jnp.int32),
        pltpu.VMEM((_N, _LANES), jnp.float32),
        pltpu.VMEM((4 * _CH,), jnp.float32),
        pltpu.SemaphoreType.DMA,
        pltpu.SemaphoreType.DMA,
    ],
)(_composite_body)


def kernel(data, images):
    bnds, coef = _prep(data, jnp.reshape(images, (_NCLS, 4)))
    tiles = _composite(bnds, coef).reshape(4, _NW * _RPT, _WPAD)
    return tiles[:, :_H, :_W]


# speedup vs baseline: 51.0796x; 1.0336x over previous
import functools

import jax
import jax.numpy as jnp
from jax import lax
from jax.experimental import pallas as pl
from jax.experimental.pallas import tpu as pltpu
from jax.experimental.pallas import tpu_sc as plsc

_H = 300
_W = 300
_N = 256
_NCLS = 64
_NC = 2
_NSUB = 16
_NW = _NC * _NSUB
_RPT = 10
_WPAD = 304
_LANES = 16
_NCHUNK = _WPAD // _LANES


def _round_half_even(x):
    f = jnp.floor(x)
    fi = f.astype(jnp.int32)
    frac = x - f
    up = (frac > 0.5) | ((frac == 0.5) & ((fi & 1) == 1))
    return fi + up.astype(jnp.int32)


def _prep_body(data_ref, img_ref, bnds_ref, coef_ref):
    data = data_ref[...]
    imgs = img_ref[...]

    x = _round_half_even(data[:, 0:1] * _H)
    y = _round_half_even(data[:, 1:2] * _W)
    h = _round_half_even(data[:, 2:3] * _H)
    w = _round_half_even(data[:, 3:4] * _W)
    x1 = x - (h >> 1)
    x2 = x + ((h + 1) >> 1)
    y1 = y - (w >> 1)
    y2 = y + ((w + 1) >> 1)
    xs = jnp.where(x1 < 0, jnp.maximum(x1 + _H, 0), x1)
    xe = jnp.clip(x2, 0, _H)
    ys = jnp.where(y1 < 0, jnp.maximum(y1 + _W, 0), y1)
    ye = jnp.clip(y2, 0, _W)

    logits = data[:, 5:]
    mx = jnp.max(logits, axis=1, keepdims=True)
    col = lax.broadcasted_iota(jnp.int32, logits.shape, 1)
    cls = jnp.min(jnp.where(logits == mx, col, _NCLS), axis=1, keepdims=True)
    onehot = (col == cls).astype(jnp.float32)
    rgba = jnp.dot(onehot, imgs, preferred_element_type=jnp.float32)
    a = rgba[:, 3:4]
    p = rgba[:, 0:3] * a
    q = 1.0 - a

    d = data[:, 4:5]
    dt = lax.transpose(d, (1, 0))
    i_col = lax.broadcasted_iota(jnp.int32, (_N, _N), 1)
    j_row = lax.broadcasted_iota(jnp.int32, (_N, _N), 0)
    before = (dt < d) | ((dt == d) & (i_col < j_row))
    rank = jnp.sum(before.astype(jnp.int32), axis=1, keepdims=True)
    perm = (lax.transpose(rank, (1, 0)) == j_row).astype(jnp.float32)

    fb = jnp.concatenate(
        [xs.astype(jnp.float32), xe.astype(jnp.float32),
         ys.astype(jnp.float32), ye.astype(jnp.float32)], axis=1)
    fc = jnp.concatenate([p, q], axis=1)
    sb = jnp.dot(perm, fb, preferred_element_type=jnp.float32)
    sc = jnp.dot(perm, fc, preferred_element_type=jnp.float32)
    pad = jnp.zeros((_N, _LANES - 4), jnp.float32)
    bnds_ref[...] = (jnp.concatenate([sb, pad], axis=1) + 0.5).astype(jnp.int32)
    coef_ref[...] = jnp.concatenate([sc, pad], axis=1)


_prep = pl.pallas_call(
    _prep_body,
    out_shape=(
        jax.ShapeDtypeStruct((_N, _LANES), jnp.int32),
        jax.ShapeDtypeStruct((_N, _LANES), jnp.float32),
    ),
)


_CH = _RPT * _WPAD


def _composite_body(bnds_hbm, coef_hbm, out_hbm, bnds_v, coef_v, canvas,
                    sem, insem):
    wid = lax.axis_index("s") * _NC + lax.axis_index("c")
    pltpu.make_async_copy(bnds_hbm, bnds_v, insem).start()
    pltpu.make_async_copy(coef_hbm, coef_v, insem).start()

    ones16 = jnp.full((_LANES,), 1.0, jnp.float32)
    iota16 = lax.broadcasted_iota(jnp.int32, (_LANES,), 0)

    @plsc.parallel_loop(0, 4 * _CH // _LANES, unroll=8)
    def _init(n):
        canvas[pl.ds(n * _LANES, _LANES)] = ones16

    for i in range(_RPT):
        pltpu.make_async_copy(
            canvas.at[pl.ds(3 * _CH + i * _WPAD, _WPAD)],
            out_hbm.at[pl.ds((3 * _NW * _RPT + wid + _NW * i) * _WPAD,
                             _WPAD)], sem).start()

    pltpu.make_async_copy(bnds_hbm, bnds_v, insem).wait()
    pltpu.make_async_copy(coef_hbm, coef_v, insem).wait()

    def _step(k, _):
        brow = bnds_v[k]
        xs = brow[0]
        xe = brow[1]
        ys = brow[2]
        ye = brow[3]
        i_lo = jnp.maximum((xs - wid + (_NW - 1)) >> 5, 0)
        i_hi = jnp.minimum((xe - wid + (_NW - 1)) >> 5, _RPT)
        t0 = ys >> 4
        t_last = (ye - 1) >> 4

        @pl.when((i_lo < i_hi) & (ys < ye))
        def _nonempty():
            crow = coef_v[k]
            qv = jnp.full((_LANES,), crow[3])
            pv = [jnp.full((_LANES,), crow[ch]) for ch in range(3)]

            def _edge(t, msk, blend):
                off0 = t * _LANES

                @plsc.parallel_loop(i_lo, i_hi, unroll=2)
                def _row(i):
                    base = off0 + i * _WPAD
                    for ch in range(3):
                        sl = pl.ds(base + ch * _CH, _LANES)
                        v = canvas[sl]
                        nv = v * qv + pv[ch] if blend else pv[ch]
                        canvas[sl] = jnp.where(msk, nv, v)

            def _do(blend):
                colv = iota16 + t0 * _LANES
                m0 = (colv >= ys) & (colv < ye)

                @pl.when(t_last == t0)
                def _single():
                    _edge(t0, m0, blend)

                @pl.when(t_last > t0)
                def _multi():
                    colv2 = iota16 + t_last * _LANES
                    mL = colv2 < ye
                    off0 = t0 * _LANES
                    offl = t_last * _LANES

                    @plsc.parallel_loop(i_lo, i_hi)
                    def _row(i):
                        base = i * _WPAD
                        for ch in range(3):
                            sl = pl.ds(base + off0 + ch * _CH, _LANES)
                            v = canvas[sl]
                            nv = v * qv + pv[ch] if blend else pv[ch]
                            canvas[sl] = jnp.where(m0, nv, v)

                        @plsc.parallel_loop(t0 + 1, t_last, unroll=4)
                        def _t(t):
                            tb = base + t * _LANES
                            for ch in range(3):
                                sl = pl.ds(tb + ch * _CH, _LANES)
                                if blend:
                                    canvas[sl] = canvas[sl] * qv + pv[ch]
                                else:
                                    canvas[sl] = pv[ch]

                        for ch in range(3):
                            sl = pl.ds(base + offl + ch * _CH, _LANES)
                            v = canvas[sl]
                            nv = v * qv + pv[ch] if blend else pv[ch]
                            canvas[sl] = jnp.where(mL, nv, v)

            opaque = crow[3] == 0.0

            @pl.when(opaque)
            def _paint():
                _do(False)

            @pl.when(jnp.logical_not(opaque))
            def _blendp():
                _do(True)

        return 0

    lax.fori_loop(0, _N, _step, 0)

    for i in range(_RPT):
        for ch in range(3):
            pltpu.make_async_copy(
                canvas.at[pl.ds(ch * _CH + i * _WPAD, _WPAD)],
                out_hbm.at[pl.ds((ch * _NW * _RPT + wid + _NW * i) * _WPAD,
                                 _WPAD)], sem).start()
    for i in range(_RPT):
        for ch in range(4):
            pltpu.make_async_copy(
                canvas.at[pl.ds(ch * _CH + i * _WPAD, _WPAD)],
                out_hbm.at[pl.ds((ch * _NW * _RPT + wid + _NW * i) * _WPAD,
                                 _WPAD)], sem).wait()


_composite = functools.partial(
    pl.kernel,
    out_type=jax.ShapeDtypeStruct((4 * _NW * _RPT * _WPAD,), jnp.float32),
    mesh=plsc.VectorSubcoreMesh(
        core_axis_name="c", subcore_axis_name="s",
        num_cores=_NC, num_subcores=_NSUB),
    scratch_types=[
        pltpu.VMEM((_N, _LANES), jnp.int32),
        pltpu.VMEM((_N, _LANES), jnp.float32),
        pltpu.VMEM((4 * _CH,), jnp.float32),
        pltpu.SemaphoreType.DMA,
        pltpu.SemaphoreType.DMA,
    ],
)(_composite_body)


def kernel(data, images):
    bnds, coef = _prep(data, jnp.reshape(images, (_NCLS, 4)))
    tiles = _composite(bnds, coef).reshape(4, _NW * _RPT, _WPAD)
    return tiles[:, :_H, :_W]
